# baseline (speedup 1.0000x reference)
import functools
import numpy as np
import jax
import jax.numpy as jnp
from jax.experimental import pallas as pl
from jax.experimental.pallas import tpu as pltpu

K = 9
NEG_BIG = -3.0e38


def _k1_body(xn_ref, xt_ref, idx_ref, *, nkeys, q):
    i = pl.program_id(1)
    xn = xn_ref[0]
    xq = xt_ref[0, :, pl.ds(i * q, q)]
    inner = jax.lax.dot_general(
        xn.astype(jnp.bfloat16), xq.astype(jnp.bfloat16),
        (((1,), (0,)), ((), ())),
        preferred_element_type=jnp.float32)
    xxk = jnp.sum(xn * xn, axis=1, keepdims=True)
    xxq = jnp.sum(xq * xq, axis=0, keepdims=True)
    cur = -((xxq - 2.0 * inner) + xxk)
    fiota = jax.lax.broadcasted_iota(
        jnp.int32, (nkeys, q), 0).astype(jnp.float32)
    for t in range(K + 1):
        idxc = jnp.argmax(cur, axis=0)[None, :]
        if t >= 1:
            idx_ref[t - 1:t, :] = idxc
        if t < K:
            fidx = idxc.astype(jnp.float32)
            cur = jnp.where(fiota == fidx, NEG_BIG, cur)


def _sc_gather(idx, x_tr):
    from jax.experimental.pallas import tpu_sc as plsc
    from jax import lax

    KN, BN = idx.shape
    B, C, N = x_tr.shape
    info = plsc.get_sparse_core_info()
    nw = info.num_cores * info.num_subcores
    ppw = BN // nw
    wpb = N // ppw
    nchunk = ppw // 16
    mesh = plsc.VectorSubcoreMesh(core_axis_name="c", subcore_axis_name="s")

    @functools.partial(
        pl.kernel, mesh=mesh,
        compiler_params=pltpu.CompilerParams(needs_layout_passes=False),
        out_type=jax.ShapeDtypeStruct((3 * KN * BN,), jnp.float32),
        scratch_types=[
            pltpu.VMEM((KN * ppw,), jnp.int32),
            pltpu.VMEM((C * N,), jnp.float32),
            pltpu.VMEM((3 * KN * ppw,), jnp.float32),
        ],
    )
    def run(idx_hbm, xtr_hbm, rel_hbm, idx_v, xb_v, rel_v):
        wid = lax.axis_index("s") * info.num_cores + lax.axis_index("c")
        base = wid * ppw
        shift = int(np.log2(wpb))
        b = lax.shift_right_logical(wid, shift)
        local = lax.bitwise_and(wid, wpb - 1) * ppw
        for kk in range(KN):
            pltpu.sync_copy(idx_hbm.at[pl.ds(kk * BN + base, ppw)],
                            idx_v.at[pl.ds(kk * ppw, ppw)])
        pltpu.sync_copy(xtr_hbm.at[pl.ds(b * C * N, C * N)], xb_v)

        def chunk(ch, _):
            off = ch * 16
            for kk in range(KN):
                iv = idx_v[pl.ds(kk * ppw + off, 16)]
                for c in range(C):
                    g = plsc.load_gather(xb_v, [iv + c * N])
                    qc = xb_v[pl.ds(c * N + local + off, 16)]
                    rel_v[pl.ds((3 * kk + c) * ppw + off, 16)] = g - qc
            return _

        lax.fori_loop(0, nchunk, chunk, 0)
        for r in range(3 * KN):
            pltpu.sync_copy(rel_v.at[pl.ds(r * ppw, ppw)],
                            rel_hbm.at[pl.ds(r * BN + base, ppw)])

    return run(idx.reshape(-1), x_tr.reshape(-1)).reshape(3 * KN, BN)


def _sorted_neighbors(rel_ref):
    two_pi = np.float32(2.0 * np.pi)
    vx, vy, vz, keys = [], [], [], []
    for kk in range(K):
        dx = rel_ref[3 * kk + 0]
        dy = rel_ref[3 * kk + 1]
        dz = rel_ref[3 * kk + 2]
        keys.append(jnp.arctan2(dy, dx) / two_pi + 0.5)
        vx.append(dx)
        vy.append(dy)
        vz.append(dz)
    for pas in range(K - 1):
        for i in range(K - 1 - pas):
            c = keys[i] > keys[i + 1]
            for arr in (keys, vx, vy, vz):
                a, b = arr[i], arr[i + 1]
                arr[i] = jnp.where(c, b, a)
                arr[i + 1] = jnp.where(c, a, b)
    return vx, vy, vz


def _features(rel_ref):
    vx, vy, vz = _sorted_neighbors(rel_ref)
    inv_sqrt3 = np.float32(1.0) / np.sqrt(np.float32(3.0))
    feats = []
    sgn = None
    for t in range(K):
        tn = (t + 1) % K
        cx = (vx[t] + vx[tn]) * 0.5
        cy = (vy[t] + vy[tn]) * 0.5
        cz = (vz[t] + vz[tn]) * 0.5
        nx = vy[t] * vz[tn] - vz[t] * vy[tn]
        ny = vz[t] * vx[tn] - vx[t] * vz[tn]
        nz = vx[t] * vy[tn] - vy[t] * vx[tn]
        nrm = jnp.sqrt(nx * nx + ny * ny + nz * nz) + 1e-6
        nx, ny, nz = nx / nrm, ny / nrm, nz / nrm
        if t == 0:
            sgn = jnp.where(nx > 0.0, 1.0, -1.0)
        nx, ny, nz = nx * sgn, ny * sgn, nz * sgn
        pos = (nx * cx + ny * cy + nz * cz) * inv_sqrt3
        feats.append((cx, cy, cz, nx, ny, nz, pos))
    return feats


def _kgeom_body(rel_ref, w1_ref, b1_ref, feat_ref, st_ref):
    i = pl.program_id(0)
    feats = _features(rel_ref)
    for kk in range(K):
        for f in range(7):
            feat_ref[7 * kk + f] = feats[kk][f]

    @pl.when(i == 0)
    def _init():
        st_ref[...] = jnp.zeros_like(st_ref)

    for j in range(10):
        b1j = b1_ref[j]
        tot = None
        sq = None
        for kk in range(K):
            z = feats[kk][0] * w1_ref[j, 0]
            for f in range(1, 7):
                z = z + feats[kk][f] * w1_ref[j, f]
            z = z + b1j
            tot = z if tot is None else tot + z
            sq = z * z if sq is None else sq + z * z
        st_ref[j:j + 1, :] += jnp.sum(tot, axis=0, keepdims=True)
        st_ref[10 + j:11 + j, :] += jnp.sum(sq, axis=0, keepdims=True)


def _layer1(feat_ref, w1_ref, b1_ref, s1_ref, t1_ref, kk):
    a1 = []
    for j in range(10):
        z = feat_ref[7 * kk] * w1_ref[j, 0]
        for f in range(1, 7):
            z = z + feat_ref[7 * kk + f] * w1_ref[j, f]
        z = z + b1_ref[j]
        a1.append(jnp.maximum(z * s1_ref[j] + t1_ref[j], 0.0))
    return a1


def _k2_body(feat_ref, w1_ref, b1_ref, s1_ref, t1_ref, w2_ref, b2_ref, st_ref):
    i = pl.program_id(0)

    @pl.when(i == 0)
    def _init():
        st_ref[...] = jnp.zeros_like(st_ref)

    tots = [None] * 10
    sqs = [None] * 10
    for kk in range(K):
        a1 = _layer1(feat_ref, w1_ref, b1_ref, s1_ref, t1_ref, kk)
        for j in range(10):
            z = a1[0] * w2_ref[j, 0]
            for f in range(1, 10):
                z = z + a1[f] * w2_ref[j, f]
            z = z + b2_ref[j]
            tots[j] = z if tots[j] is None else tots[j] + z
            sqs[j] = z * z if sqs[j] is None else sqs[j] + z * z
    for j in range(10):
        st_ref[j:j + 1, :] += jnp.sum(tots[j], axis=0, keepdims=True)
        st_ref[10 + j:11 + j, :] += jnp.sum(sqs[j], axis=0, keepdims=True)


def _k3_body(feat_ref, w1_ref, b1_ref, s1_ref, t1_ref, w2_ref, b2_ref,
             s2_ref, t2_ref, out_ref):
    best = [None] * 10
    for kk in range(K):
        a1 = _layer1(feat_ref, w1_ref, b1_ref, s1_ref, t1_ref, kk)
        for j in range(10):
            z = a1[0] * w2_ref[j, 0]
            for f in range(1, 10):
                z = z + a1[f] * w2_ref[j, f]
            z = jnp.maximum((z + b2_ref[j]) * s2_ref[j] + t2_ref[j], 0.0)
            best[j] = z if best[j] is None else jnp.maximum(best[j], z)
    for j in range(10):
        out_ref[j] = best[j]


def _smem_spec():
    return pl.BlockSpec(memory_space=pltpu.MemorySpace.SMEM)


@jax.jit
def _run(x, W1, b1, g1, be1, W2, b2, g2, be2):
    B, N, _ = x.shape
    BN = B * N
    Q = 1024
    nq = N // Q
    x_tr = jnp.transpose(x, (0, 2, 1))

    idx = pl.pallas_call(
        functools.partial(_k1_body, nkeys=N, q=Q),
        grid=(B, nq),
        in_specs=[
            pl.BlockSpec((1, N, 3), lambda b, i: (b, 0, 0)),
            pl.BlockSpec((1, 3, N), lambda b, i: (b, 0, 0)),
        ],
        out_specs=pl.BlockSpec((K, Q), lambda b, i: (0, b * nq + i)),
        out_shape=jax.ShapeDtypeStruct((K, BN), jnp.int32),
    )(x, x_tr)

    rel = _sc_gather(idx, x_tr)

    LN = 128
    NC = BN // LN
    S = 16
    rel3 = rel.reshape(3 * K, NC, LN)

    feat, st1 = pl.pallas_call(
        _kgeom_body,
        grid=(NC // S,),
        in_specs=[
            pl.BlockSpec((3 * K, S, LN), lambda i: (0, i, 0)),
            _smem_spec(),
            _smem_spec(),
        ],
        out_specs=[
            pl.BlockSpec((7 * K, S, LN), lambda i: (0, i, 0)),
            pl.BlockSpec((20, LN), lambda i: (0, 0)),
        ],
        out_shape=[
            jax.ShapeDtypeStruct((7 * K, NC, LN), jnp.float32),
            jax.ShapeDtypeStruct((20, LN), jnp.float32),
        ],
    )(rel3, W1, b1)

    M = np.float32(BN * K)
    m1 = jnp.sum(st1[0:10], axis=-1) / M
    v1 = jnp.sum(st1[10:20], axis=-1) / M - m1 * m1
    s1 = g1 / jnp.sqrt(v1 + 1e-5)
    t1 = be1 - m1 * s1

    st2 = pl.pallas_call(
        _k2_body,
        grid=(NC // S,),
        in_specs=[pl.BlockSpec((7 * K, S, LN), lambda i: (0, i, 0))]
        + [_smem_spec()] * 6,
        out_specs=pl.BlockSpec((20, LN), lambda i: (0, 0)),
        out_shape=jax.ShapeDtypeStruct((20, LN), jnp.float32),
    )(feat, W1, b1, s1, t1, W2, b2)

    m2 = jnp.sum(st2[0:10], axis=-1) / M
    v2 = jnp.sum(st2[10:20], axis=-1) / M - m2 * m2
    s2 = g2 / jnp.sqrt(v2 + 1e-5)
    t2 = be2 - m2 * s2

    out_t = pl.pallas_call(
        _k3_body,
        grid=(NC // S,),
        in_specs=[pl.BlockSpec((7 * K, S, LN), lambda i: (0, i, 0))]
        + [_smem_spec()] * 8,
        out_specs=pl.BlockSpec((10, S, LN), lambda i: (0, i, 0)),
        out_shape=jax.ShapeDtypeStruct((10, NC, LN), jnp.float32),
    )(feat, W1, b1, s1, t1, W2, b2, s2, t2)

    return jnp.transpose(out_t.reshape(10, B, N), (1, 2, 0))


def kernel(x, W1, b1, g1, be1, W2, b2, g2, be2, k):
    return _run(x, W1, b1, g1, be1, W2, b2, g2, be2)


# speedup vs baseline: 1.0150x; 1.0150x over previous
import functools
import numpy as np
import jax
import jax.numpy as jnp
from jax.experimental import pallas as pl
from jax.experimental.pallas import tpu as pltpu

K = 9
NEG_BIG = -3.0e38


def _k1_body(xn_ref, xt_ref, idx_ref, *, nkeys, q):
    i = pl.program_id(1)
    xn = xn_ref[0]
    xq = xt_ref[0, :, pl.ds(i * q, q)]
    inner = jax.lax.dot_general(
        xn.astype(jnp.bfloat16), xq.astype(jnp.bfloat16),
        (((1,), (0,)), ((), ())),
        preferred_element_type=jnp.float32)
    xxk = jnp.sum(xn * xn, axis=1, keepdims=True)
    xxq = jnp.sum(xq * xq, axis=0, keepdims=True)
    cur = -((xxq - 2.0 * inner) + xxk)
    fiota = jax.lax.broadcasted_iota(
        jnp.int32, (nkeys, q), 0).astype(jnp.float32)
    for t in range(K + 1):
        idxc = jnp.argmax(cur, axis=0)[None, :]
        if t >= 1:
            idx_ref[t - 1:t, :] = idxc
        if t < K:
            fidx = idxc.astype(jnp.float32)
            cur = jnp.where(fiota == fidx, NEG_BIG, cur)


def _sc_gather(idx, x_tr):
    from jax.experimental.pallas import tpu_sc as plsc
    from jax import lax

    KN, BN = idx.shape
    B, C, N = x_tr.shape
    info = plsc.get_sparse_core_info()
    nw = info.num_cores * info.num_subcores
    ppw = BN // nw
    wpb = N // ppw
    nchunk = ppw // 16
    mesh = plsc.VectorSubcoreMesh(core_axis_name="c", subcore_axis_name="s")

    @functools.partial(
        pl.kernel, mesh=mesh,
        compiler_params=pltpu.CompilerParams(needs_layout_passes=False),
        out_type=jax.ShapeDtypeStruct((3 * KN * BN,), jnp.float32),
        scratch_types=[
            pltpu.VMEM((KN * ppw,), jnp.int32),
            pltpu.VMEM((C * N,), jnp.float32),
            pltpu.VMEM((3 * KN * ppw,), jnp.float32),
        ],
    )
    def run(idx_hbm, xtr_hbm, rel_hbm, idx_v, xb_v, rel_v):
        wid = lax.axis_index("s") * info.num_cores + lax.axis_index("c")
        base = wid * ppw
        shift = int(np.log2(wpb))
        b = lax.shift_right_logical(wid, shift)
        local = lax.bitwise_and(wid, wpb - 1) * ppw
        for kk in range(KN):
            pltpu.sync_copy(idx_hbm.at[pl.ds(kk * BN + base, ppw)],
                            idx_v.at[pl.ds(kk * ppw, ppw)])
        pltpu.sync_copy(xtr_hbm.at[pl.ds(b * C * N, C * N)], xb_v)

        def chunk(ch, _):
            off = ch * 16
            for kk in range(KN):
                iv = idx_v[pl.ds(kk * ppw + off, 16)]
                for c in range(C):
                    g = plsc.load_gather(xb_v, [iv + c * N])
                    qc = xb_v[pl.ds(c * N + local + off, 16)]
                    rel_v[pl.ds((3 * kk + c) * ppw + off, 16)] = g - qc
            return _

        lax.fori_loop(0, nchunk, chunk, 0)
        for r in range(3 * KN):
            pltpu.sync_copy(rel_v.at[pl.ds(r * ppw, ppw)],
                            rel_hbm.at[pl.ds(r * BN + base, ppw)])

    return run(idx.reshape(-1), x_tr.reshape(-1)).reshape(3 * KN, BN)


def _sorted_neighbors(rel_ref):
    two_pi = np.float32(2.0 * np.pi)
    vx, vy, vz, keys = [], [], [], []
    for kk in range(K):
        dx = rel_ref[3 * kk + 0]
        dy = rel_ref[3 * kk + 1]
        dz = rel_ref[3 * kk + 2]
        keys.append(jnp.arctan2(dy, dx) / two_pi + 0.5)
        vx.append(dx)
        vy.append(dy)
        vz.append(dz)
    for pas in range(K - 1):
        for i in range(K - 1 - pas):
            c = keys[i] > keys[i + 1]
            for arr in (keys, vx, vy, vz):
                a, b = arr[i], arr[i + 1]
                arr[i] = jnp.where(c, b, a)
                arr[i + 1] = jnp.where(c, a, b)
    return vx, vy, vz


def _features(rel_ref):
    vx, vy, vz = _sorted_neighbors(rel_ref)
    inv_sqrt3 = np.float32(1.0) / np.sqrt(np.float32(3.0))
    feats = []
    sgn = None
    for t in range(K):
        tn = (t + 1) % K
        cx = (vx[t] + vx[tn]) * 0.5
        cy = (vy[t] + vy[tn]) * 0.5
        cz = (vz[t] + vz[tn]) * 0.5
        nx = vy[t] * vz[tn] - vz[t] * vy[tn]
        ny = vz[t] * vx[tn] - vx[t] * vz[tn]
        nz = vx[t] * vy[tn] - vy[t] * vx[tn]
        nrm = jnp.sqrt(nx * nx + ny * ny + nz * nz) + 1e-6
        nx, ny, nz = nx / nrm, ny / nrm, nz / nrm
        if t == 0:
            sgn = jnp.where(nx > 0.0, 1.0, -1.0)
        nx, ny, nz = nx * sgn, ny * sgn, nz * sgn
        pos = (nx * cx + ny * cy + nz * cz) * inv_sqrt3
        feats.append((cx, cy, cz, nx, ny, nz, pos))
    return feats


def _kgeom_body(rel_ref, w1_ref, b1_ref, feat_ref, st_ref):
    i = pl.program_id(0)
    feats = _features(rel_ref)
    for kk in range(K):
        for f in range(7):
            feat_ref[7 * kk + f] = feats[kk][f]

    @pl.when(i == 0)
    def _init():
        st_ref[...] = jnp.zeros_like(st_ref)

    for j in range(10):
        b1j = b1_ref[j]
        tot = None
        sq = None
        for kk in range(K):
            z = feats[kk][0] * w1_ref[j, 0]
            for f in range(1, 7):
                z = z + feats[kk][f] * w1_ref[j, f]
            z = z + b1j
            tot = z if tot is None else tot + z
            sq = z * z if sq is None else sq + z * z
        st_ref[j:j + 1, :] += jnp.sum(tot, axis=0, keepdims=True)
        st_ref[10 + j:11 + j, :] += jnp.sum(sq, axis=0, keepdims=True)


def _layer1(feat_ref, w1_ref, b1_ref, s1_ref, t1_ref, kk):
    a1 = []
    for j in range(10):
        z = feat_ref[7 * kk] * w1_ref[j, 0]
        for f in range(1, 7):
            z = z + feat_ref[7 * kk + f] * w1_ref[j, f]
        z = z + b1_ref[j]
        a1.append(jnp.maximum(z * s1_ref[j] + t1_ref[j], 0.0))
    return a1


def _k2_body(feat_ref, w1_ref, b1_ref, s1_ref, t1_ref, w2_ref, b2_ref, st_ref):
    i = pl.program_id(0)

    @pl.when(i == 0)
    def _init():
        st_ref[...] = jnp.zeros_like(st_ref)

    tots = [None] * 10
    sqs = [None] * 10
    for kk in range(K):
        a1 = _layer1(feat_ref, w1_ref, b1_ref, s1_ref, t1_ref, kk)
        for j in range(10):
            z = a1[0] * w2_ref[j, 0]
            for f in range(1, 10):
                z = z + a1[f] * w2_ref[j, f]
            z = z + b2_ref[j]
            tots[j] = z if tots[j] is None else tots[j] + z
            sqs[j] = z * z if sqs[j] is None else sqs[j] + z * z
    for j in range(10):
        st_ref[j:j + 1, :] += jnp.sum(tots[j], axis=0, keepdims=True)
        st_ref[10 + j:11 + j, :] += jnp.sum(sqs[j], axis=0, keepdims=True)


def _k3_body(feat_ref, w1_ref, b1_ref, s1_ref, t1_ref, w2_ref, b2_ref,
             s2_ref, t2_ref, out_ref):
    best = [None] * 10
    for kk in range(K):
        a1 = _layer1(feat_ref, w1_ref, b1_ref, s1_ref, t1_ref, kk)
        for j in range(10):
            z = a1[0] * w2_ref[j, 0]
            for f in range(1, 10):
                z = z + a1[f] * w2_ref[j, f]
            z = jnp.maximum((z + b2_ref[j]) * s2_ref[j] + t2_ref[j], 0.0)
            best[j] = z if best[j] is None else jnp.maximum(best[j], z)
    for j in range(10):
        out_ref[j] = best[j]


def _smem_spec():
    return pl.BlockSpec(memory_space=pltpu.MemorySpace.SMEM)


@jax.jit
def _run(x, W1, b1, g1, be1, W2, b2, g2, be2):
    B, N, _ = x.shape
    BN = B * N
    Q = 256
    nq = N // Q
    x_tr = jnp.transpose(x, (0, 2, 1))

    idx = pl.pallas_call(
        functools.partial(_k1_body, nkeys=N, q=Q),
        grid=(B, nq),
        in_specs=[
            pl.BlockSpec((1, N, 3), lambda b, i: (b, 0, 0)),
            pl.BlockSpec((1, 3, N), lambda b, i: (b, 0, 0)),
        ],
        out_specs=pl.BlockSpec((K, Q), lambda b, i: (0, b * nq + i)),
        out_shape=jax.ShapeDtypeStruct((K, BN), jnp.int32),
    )(x, x_tr)

    rel = _sc_gather(idx, x_tr)

    LN = 128
    NC = BN // LN
    S = 16
    rel3 = rel.reshape(3 * K, NC, LN)

    feat, st1 = pl.pallas_call(
        _kgeom_body,
        grid=(NC // S,),
        in_specs=[
            pl.BlockSpec((3 * K, S, LN), lambda i: (0, i, 0)),
            _smem_spec(),
            _smem_spec(),
        ],
        out_specs=[
            pl.BlockSpec((7 * K, S, LN), lambda i: (0, i, 0)),
            pl.BlockSpec((20, LN), lambda i: (0, 0)),
        ],
        out_shape=[
            jax.ShapeDtypeStruct((7 * K, NC, LN), jnp.float32),
            jax.ShapeDtypeStruct((20, LN), jnp.float32),
        ],
    )(rel3, W1, b1)

    M = np.float32(BN * K)
    m1 = jnp.sum(st1[0:10], axis=-1) / M
    v1 = jnp.sum(st1[10:20], axis=-1) / M - m1 * m1
    s1 = g1 / jnp.sqrt(v1 + 1e-5)
    t1 = be1 - m1 * s1

    st2 = pl.pallas_call(
        _k2_body,
        grid=(NC // S,),
        in_specs=[pl.BlockSpec((7 * K, S, LN), lambda i: (0, i, 0))]
        + [_smem_spec()] * 6,
        out_specs=pl.BlockSpec((20, LN), lambda i: (0, 0)),
        out_shape=jax.ShapeDtypeStruct((20, LN), jnp.float32),
    )(feat, W1, b1, s1, t1, W2, b2)

    m2 = jnp.sum(st2[0:10], axis=-1) / M
    v2 = jnp.sum(st2[10:20], axis=-1) / M - m2 * m2
    s2 = g2 / jnp.sqrt(v2 + 1e-5)
    t2 = be2 - m2 * s2

    out_t = pl.pallas_call(
        _k3_body,
        grid=(NC // S,),
        in_specs=[pl.BlockSpec((7 * K, S, LN), lambda i: (0, i, 0))]
        + [_smem_spec()] * 8,
        out_specs=pl.BlockSpec((10, S, LN), lambda i: (0, i, 0)),
        out_shape=jax.ShapeDtypeStruct((10, NC, LN), jnp.float32),
    )(feat, W1, b1, s1, t1, W2, b2, s2, t2)

    return jnp.transpose(out_t.reshape(10, B, N), (1, 2, 0))


def kernel(x, W1, b1, g1, be1, W2, b2, g2, be2, k):
    return _run(x, W1, b1, g1, be1, W2, b2, g2, be2)
